# baseline (speedup 1.0000x reference)
import functools

import jax
import jax.numpy as jnp
from jax import lax
from jax.experimental import pallas as pl
from jax.experimental.pallas import tpu as pltpu
from jax.experimental.pallas import tpu_sc as plsc

NC = 2
NS = 16
NW = NC * NS

N = 10000
NP = 10240
NPR = NP // 128
E = 320000
EPT = E // NW
CHUNK = 128
EP = NW * 80 * CHUNK
NITER = EP // (NW * CHUNK)
PH = 2
CPP = NITER // PH
PPP = CPP // 2
RPT = NP // NS
NB = 128
NGRID = (N + NB - 1) // NB


def _mesh():
    return plsc.VectorSubcoreMesh(
        core_axis_name="c", subcore_axis_name="s", num_cores=NC, num_subcores=NS
    )


def _sc_hist(col):
    @functools.partial(
        pl.kernel,
        out_type=jax.ShapeDtypeStruct((NW, NP), jnp.float32),
        mesh=_mesh(),
        compiler_params=pltpu.CompilerParams(needs_layout_passes=False),
        scratch_types=[
            pltpu.VMEM((EPT,), jnp.int32),
            pltpu.VMEM((NP,), jnp.float32),
        ],
    )
    def k(col_hbm, out_hbm, col_v, hist_v):
        cid = lax.axis_index("c")
        sid = lax.axis_index("s")
        wid = cid * NS + sid

        def zbody(i, carry):
            hist_v[pl.ds(i * 16, 16)] = jnp.zeros((16,), jnp.float32)
            return carry

        lax.fori_loop(0, NP // 16, zbody, 0)
        pltpu.sync_copy(col_hbm.at[pl.ds(wid * EPT, EPT)], col_v)
        ones = jnp.ones((16,), jnp.float32)

        def hbody(i, carry):
            idx = col_v[pl.ds(i * 16, 16)]
            plsc.addupdate_scatter(hist_v, [idx], ones)
            return carry

        lax.fori_loop(0, EPT // 16, hbody, 0)
        pltpu.sync_copy(hist_v, out_hbm.at[wid])

    return k(col)


def _sc_scatter(row3d, col3d, g):
    @functools.partial(
        pl.kernel,
        out_type=jax.ShapeDtypeStruct((NC, NP, 128), jnp.float32),
        mesh=_mesh(),
        compiler_params=pltpu.CompilerParams(needs_layout_passes=False),
        scratch_types=[
            pltpu.VMEM((CPP, CHUNK), jnp.int32),
            pltpu.VMEM((CPP, CHUNK), jnp.int32),
            pltpu.VMEM((CHUNK, 128), jnp.float32),
            pltpu.VMEM((CHUNK, 128), jnp.float32),
            pltpu.VMEM_SHARED((NP, 128), jnp.float32),
            pltpu.SemaphoreType.DMA,
            pltpu.SemaphoreType.DMA,
        ],
    )
    def k(row_hbm, col_hbm, g_hbm, out_hbm, ridx, cidx, buf0, buf1, acc, s0, s1):
        cid = lax.axis_index("c")
        sid = lax.axis_index("s")
        wid = cid * NS + sid

        def zbody(i, carry):
            r = i // 8
            cc = (i % 8) * 16
            buf0[r, pl.ds(cc, 16)] = jnp.zeros((16,), jnp.float32)
            return carry

        lax.fori_loop(0, CHUNK * 8, zbody, 0)
        for b in range(RPT // CHUNK):
            pltpu.sync_copy(buf0, acc.at[pl.ds(sid * RPT + b * CHUNK, CHUNK), :])
        plsc.subcore_barrier()

        def body(j, carry):
            i0 = 2 * j
            i1 = 2 * j + 1
            pltpu.make_async_copy(g_hbm.at[ridx.at[i0]], buf0, s0).wait()
            pltpu.async_copy(g_hbm.at[ridx.at[i1]], buf1, s1)
            pltpu.sync_copy(buf0, acc.at[cidx.at[i0]], add=True)
            pltpu.make_async_copy(g_hbm.at[ridx.at[i1]], buf1, s1).wait()

            @pl.when(j + 1 < PPP)
            def _():
                pltpu.async_copy(g_hbm.at[ridx.at[i0 + 2]], buf0, s0)

            pltpu.sync_copy(buf1, acc.at[cidx.at[i1]], add=True)
            return carry

        for ph in range(PH):
            pltpu.sync_copy(row_hbm.at[wid, pl.ds(ph * CPP, CPP)], ridx)
            pltpu.sync_copy(col_hbm.at[wid, pl.ds(ph * CPP, CPP)], cidx)
            pltpu.async_copy(g_hbm.at[ridx.at[0]], buf0, s0)
            lax.fori_loop(0, PPP, body, 0)
        plsc.subcore_barrier()
        pltpu.sync_copy(
            acc.at[pl.ds(sid * RPT, RPT), :],
            out_hbm.at[cid, pl.ds(sid * RPT, RPT), :],
        )

    return k(row3d, col3d, g)


def _diag(dinv):
    ii = lax.broadcasted_iota(jnp.int32, (128, 128), 0)
    jj = lax.broadcasted_iota(jnp.int32, (128, 128), 1)
    return jnp.where(ii == jj, jnp.broadcast_to(dinv, (128, 128)), 0.0)


def _rowmask(i, v):
    rid = i * NB + lax.broadcasted_iota(jnp.int32, v.shape, 0)
    return jnp.where(rid < N, v, 0.0)


def _tc_g(parts, x, w):
    def body(p_ref, x_ref, w_ref, o_ref):
        i = pl.program_id(0)
        p = p_ref[:, pl.ds(i * NB, NB)]
        d = jnp.sum(p, axis=0, keepdims=True) + 1.0
        dinv = lax.rsqrt(d)
        h = jnp.dot(
            _rowmask(i, x_ref[...]), w_ref[...], preferred_element_type=jnp.float32
        )
        o_ref[...] = jnp.dot(_diag(dinv), h, preferred_element_type=jnp.float32)

    return pl.pallas_call(
        body,
        grid=(NGRID,),
        in_specs=[
            pl.BlockSpec((NW, NP), lambda i: (0, 0)),
            pl.BlockSpec((NB, 128), lambda i: (i, 0)),
            pl.BlockSpec((128, 128), lambda i: (0, 0)),
        ],
        out_specs=pl.BlockSpec((NB, 128), lambda i: (i, 0)),
        out_shape=jax.ShapeDtypeStruct((N, 128), jnp.float32),
    )(parts, x, w)


def _tc_out(sparts, g, degparts, bc, pa, wl, bl):
    def body(s_ref, g_ref, p_ref, bc_ref, pa_ref, wl_ref, bl_ref, o_ref):
        i = pl.program_id(0)
        p = p_ref[:, pl.ds(i * NB, NB)]
        d = jnp.sum(p, axis=0, keepdims=True) + 1.0
        dinv = lax.rsqrt(d)
        s = _rowmask(i, s_ref[0] + s_ref[1] + g_ref[...])
        agg = jnp.dot(_diag(dinv), s, preferred_element_type=jnp.float32)
        agg = agg + bc_ref[...]
        a = pa_ref[0, 0]
        act = jnp.where(agg > 0, agg, a * agg)
        o_ref[...] = (
            jnp.dot(act, wl_ref[...], preferred_element_type=jnp.float32) + bl_ref[...]
        )

    return pl.pallas_call(
        body,
        grid=(NGRID,),
        in_specs=[
            pl.BlockSpec((NC, NB, 128), lambda i: (0, i, 0)),
            pl.BlockSpec((NB, 128), lambda i: (i, 0)),
            pl.BlockSpec((NW, NP), lambda i: (0, 0)),
            pl.BlockSpec((1, 128), lambda i: (0, 0)),
            pl.BlockSpec((1, 1), lambda i: (0, 0)),
            pl.BlockSpec((128, 128), lambda i: (0, 0)),
            pl.BlockSpec((1, 128), lambda i: (0, 0)),
        ],
        out_specs=pl.BlockSpec((NB, 128), lambda i: (i, 0)),
        out_shape=jax.ShapeDtypeStruct((N, 128), jnp.float32),
    )(sparts, g, degparts, bc, pa, wl, bl)


def kernel(x, adj, W_conv, b_conv, prelu_a, W_lin, b_lin):
    row = adj[0]
    col = adj[1]
    degparts = _sc_hist(col)
    g = _tc_g(degparts, x, W_conv)
    pad = EP - E
    rowp = jnp.concatenate([row, jnp.zeros((pad,), row.dtype)])
    colp = jnp.concatenate([col, jnp.full((pad,), NP - 1, col.dtype)])
    sparts = _sc_scatter(
        rowp.reshape(NW, NITER, CHUNK), colp.reshape(NW, NITER, CHUNK), g
    )
    return _tc_out(
        sparts,
        g,
        degparts,
        b_conv.reshape(1, 128),
        jnp.asarray(prelu_a, jnp.float32).reshape(1, 1),
        W_lin,
        b_lin.reshape(1, 128),
    )


# speedup vs baseline: 2.5904x; 2.5904x over previous
import functools

import jax
import jax.numpy as jnp
from jax import lax
from jax.experimental import pallas as pl
from jax.experimental.pallas import tpu as pltpu
from jax.experimental.pallas import tpu_sc as plsc

NC = 2
NS = 16
NW = NC * NS

N = 10000
NP = 10240
NPR = NP // 128
E = 320000
EPT = E // NW
CHUNK = 128
EP = NW * 80 * CHUNK
NITER = EP // (NW * CHUNK)
PH = 2
CPP = NITER // PH
PPP = CPP // 2
RPT = NP // NS
NB = 128
NGRID = (N + NB - 1) // NB


def _mesh():
    return plsc.VectorSubcoreMesh(
        core_axis_name="c", subcore_axis_name="s", num_cores=NC, num_subcores=NS
    )


def _sc_hist(col):
    @functools.partial(
        pl.kernel,
        out_type=jax.ShapeDtypeStruct((NW, NP), jnp.float32),
        mesh=_mesh(),
        compiler_params=pltpu.CompilerParams(needs_layout_passes=False),
        scratch_types=[
            pltpu.VMEM((EPT,), jnp.int32),
            pltpu.VMEM((NP,), jnp.float32),
        ],
    )
    def k(col_hbm, out_hbm, col_v, hist_v):
        cid = lax.axis_index("c")
        sid = lax.axis_index("s")
        wid = cid * NS + sid

        def zbody(i, carry):
            hist_v[pl.ds(i * 16, 16)] = jnp.zeros((16,), jnp.float32)
            return carry

        lax.fori_loop(0, NP // 16, zbody, 0)
        pltpu.sync_copy(col_hbm.at[pl.ds(wid * EPT, EPT)], col_v)
        ones = jnp.ones((16,), jnp.float32)

        def hbody(i, carry):
            idx = col_v[pl.ds(i * 16, 16)]
            plsc.addupdate_scatter(hist_v, [idx], ones)
            return carry

        lax.fori_loop(0, EPT // 16, hbody, 0)
        pltpu.sync_copy(hist_v, out_hbm.at[wid])

    return k(col)


def _sc_scatter(row3d, col3d, g):
    @functools.partial(
        pl.kernel,
        out_type=jax.ShapeDtypeStruct((NC, NP, 128), jnp.float32),
        mesh=_mesh(),
        compiler_params=pltpu.CompilerParams(needs_layout_passes=False),
        scratch_types=[
            pltpu.VMEM((CPP, CHUNK), jnp.int32),
            pltpu.VMEM((CPP, CHUNK), jnp.int32),
            pltpu.VMEM((CHUNK, 128), jnp.float32),
            pltpu.VMEM((CHUNK, 128), jnp.float32),
            pltpu.VMEM_SHARED((NP, 128), jnp.float32),
            pltpu.SemaphoreType.DMA,
            pltpu.SemaphoreType.DMA,
        ],
    )
    def k(row_hbm, col_hbm, g_hbm, out_hbm, ridx, cidx, buf0, buf1, acc, s0, s1):
        cid = lax.axis_index("c")
        sid = lax.axis_index("s")
        wid = cid * NS + sid

        def zbody(i, carry):
            r = i // 8
            cc = (i % 8) * 16
            buf0[r, pl.ds(cc, 16)] = jnp.zeros((16,), jnp.float32)
            return carry

        lax.fori_loop(0, CHUNK * 8, zbody, 0)
        for b in range(RPT // CHUNK):
            pltpu.sync_copy(buf0, acc.at[pl.ds(sid * RPT + b * CHUNK, CHUNK), :])
        plsc.subcore_barrier()

        def body(j, carry):
            i0 = 2 * j
            i1 = 2 * j + 1
            pltpu.make_async_copy(g_hbm.at[ridx.at[i0]], buf0, s0).wait()
            pltpu.async_copy(g_hbm.at[ridx.at[i1]], buf1, s1)
            pltpu.sync_copy(buf0, acc.at[cidx.at[i0]], add=True)
            pltpu.make_async_copy(g_hbm.at[ridx.at[i1]], buf1, s1).wait()

            @pl.when(j + 1 < PPP)
            def _():
                pltpu.async_copy(g_hbm.at[ridx.at[i0 + 2]], buf0, s0)

            pltpu.sync_copy(buf1, acc.at[cidx.at[i1]], add=True)
            return carry

        for ph in range(PH):
            pltpu.sync_copy(row_hbm.at[wid, pl.ds(ph * CPP, CPP)], ridx)
            pltpu.sync_copy(col_hbm.at[wid, pl.ds(ph * CPP, CPP)], cidx)
            pltpu.async_copy(g_hbm.at[ridx.at[0]], buf0, s0)
            lax.fori_loop(0, PPP, body, 0)
        plsc.subcore_barrier()
        pltpu.sync_copy(
            acc.at[pl.ds(sid * RPT, RPT), :],
            out_hbm.at[cid, pl.ds(sid * RPT, RPT), :],
        )

    return k(row3d, col3d, g)


def _diag(dinv):
    ii = lax.broadcasted_iota(jnp.int32, (128, 128), 0)
    jj = lax.broadcasted_iota(jnp.int32, (128, 128), 1)
    return jnp.where(ii == jj, jnp.broadcast_to(dinv, (128, 128)), 0.0)


def _rowmask(i, v):
    rid = i * NB + lax.broadcasted_iota(jnp.int32, v.shape, 0)
    return jnp.where(rid < N, v, 0.0)


def _tc_g(parts, x, w):
    def body(p_ref, x_ref, w_ref, o_ref):
        i = pl.program_id(0)
        p = p_ref[:, pl.ds(i * NB, NB)]
        d = jnp.sum(p, axis=0, keepdims=True) + 1.0
        dinv = lax.rsqrt(d)
        h = jnp.dot(
            _rowmask(i, x_ref[...]), w_ref[...], preferred_element_type=jnp.float32
        )
        o_ref[...] = jnp.dot(_diag(dinv), h, preferred_element_type=jnp.float32)

    return pl.pallas_call(
        body,
        grid=(NGRID,),
        in_specs=[
            pl.BlockSpec((NW, NP), lambda i: (0, 0)),
            pl.BlockSpec((NB, 128), lambda i: (i, 0)),
            pl.BlockSpec((128, 128), lambda i: (0, 0)),
        ],
        out_specs=pl.BlockSpec((NB, 128), lambda i: (i, 0)),
        out_shape=jax.ShapeDtypeStruct((N, 128), jnp.float32),
    )(parts, x, w)


def _tc_out(sparts, g, degparts, bc, pa, wl, bl):
    def body(s_ref, g_ref, p_ref, bc_ref, pa_ref, wl_ref, bl_ref, o_ref):
        i = pl.program_id(0)
        p = p_ref[:, pl.ds(i * NB, NB)]
        d = jnp.sum(p, axis=0, keepdims=True) + 1.0
        dinv = lax.rsqrt(d)
        s = _rowmask(i, s_ref[0] + s_ref[1] + g_ref[...])
        agg = jnp.dot(_diag(dinv), s, preferred_element_type=jnp.float32)
        agg = agg + bc_ref[...]
        a = pa_ref[0, 0]
        act = jnp.where(agg > 0, agg, a * agg)
        o_ref[...] = (
            jnp.dot(act, wl_ref[...], preferred_element_type=jnp.float32) + bl_ref[...]
        )

    return pl.pallas_call(
        body,
        grid=(NGRID,),
        in_specs=[
            pl.BlockSpec((NC, NB, 128), lambda i: (0, i, 0)),
            pl.BlockSpec((NB, 128), lambda i: (i, 0)),
            pl.BlockSpec((NW, NP), lambda i: (0, 0)),
            pl.BlockSpec((1, 128), lambda i: (0, 0)),
            pl.BlockSpec((1, 1), lambda i: (0, 0)),
            pl.BlockSpec((128, 128), lambda i: (0, 0)),
            pl.BlockSpec((1, 128), lambda i: (0, 0)),
        ],
        out_specs=pl.BlockSpec((NB, 128), lambda i: (i, 0)),
        out_shape=jax.ShapeDtypeStruct((N, 128), jnp.float32),
    )(sparts, g, degparts, bc, pa, wl, bl)


def kernel(x, adj, W_conv, b_conv, prelu_a, W_lin, b_lin):
    row = adj[0]
    col = adj[1]
    degparts = _sc_hist(col)
    g = _tc_g(degparts, x, W_conv)
    pad = EP - E
    ar = jnp.arange(pad, dtype=row.dtype)
    rowp = jnp.concatenate([row, (ar * 131) % N])
    colp = jnp.concatenate([col, N + ar % (NP - N)])
    sparts = _sc_scatter(
        rowp.reshape(NW, NITER, CHUNK), colp.reshape(NW, NITER, CHUNK), g
    )
    return _tc_out(
        sparts,
        g,
        degparts,
        b_conv.reshape(1, 128),
        jnp.asarray(prelu_a, jnp.float32).reshape(1, 1),
        W_lin,
        b_lin.reshape(1, 128),
    )
